# baseline (speedup 1.0000x reference)
import jax
import jax.numpy as jnp
from jax.experimental import pallas as pl
from jax.experimental.pallas import tpu as pltpu

_BM = 200
_NBUF = 5


def _sgc_kernel(a_hbm, x_ref, w_ref, b_ref, o_ref, bufs, y_ref, sems):
    n = x_ref.shape[0]
    nblk = n // _BM
    nsteps = 2 * nblk
    p1_reuse_end = nblk + _NBUF

    def block_of(s):
        return jnp.where(
            s < nblk, s,
            jnp.where(s < p1_reuse_end, s - _NBUF, s - p1_reuse_end))

    def dmas(s):
        blk = block_of(s)
        slot = jax.lax.rem(s, _NBUF)
        h0 = 104
        h1 = _BM - h0
        return [
            pltpu.make_async_copy(
                a_hbm.at[pl.ds(blk * _BM, h0), :],
                bufs.at[slot, pl.ds(0, h0), :], sems.at[slot, 0]),
            pltpu.make_async_copy(
                a_hbm.at[pl.ds(blk * _BM + h0, h1), :],
                bufs.at[slot, pl.ds(h0, h1), :], sems.at[slot, 1]),
        ]

    def needs_dma(s):
        return jnp.logical_or(s < nblk, s >= p1_reuse_end)

    for s in range(_NBUF):
        for c in dmas(s):
            c.start()

    def step(s, _):
        slot = jax.lax.rem(s, _NBUF)

        @pl.when(needs_dma(s))
        def _():
            for c in dmas(s):
                c.wait()

        a = bufs.at[slot][...]
        base = block_of(s) * _BM

        @pl.when(s < nblk)
        def _():
            y_ref[pl.ds(base, _BM), :] = jax.lax.dot_general(
                a, x_ref[...], (((1,), (0,)), ((), ())),
                preferred_element_type=jnp.float32)

        @pl.when(s >= nblk)
        def _():
            h = jax.lax.dot_general(
                a, y_ref[...], (((1,), (0,)), ((), ())),
                preferred_element_type=jnp.float32)
            logits = jax.lax.dot_general(
                h, w_ref[...], (((1,), (1,)), ((), ())),
                preferred_element_type=jnp.float32)
            logits = logits + b_ref[...]
            m = jnp.max(logits, axis=1, keepdims=True)
            shifted = logits - m
            lse = jnp.log(jnp.sum(jnp.exp(shifted), axis=1, keepdims=True))
            o_ref[pl.ds(base, _BM), :] = shifted - lse

        nxt = s + _NBUF

        @pl.when(jnp.logical_and(nxt < nsteps, needs_dma(nxt)))
        def _():
            for c in dmas(nxt):
                c.start()

        return _

    jax.lax.fori_loop(0, nsteps, step, None)


def kernel(x, adj_norm, W, b):
    n, nfeat = x.shape
    nclass = W.shape[0]

    return pl.pallas_call(
        _sgc_kernel,
        in_specs=[
            pl.BlockSpec(memory_space=pl.ANY),
            pl.BlockSpec(memory_space=pltpu.MemorySpace.VMEM),
            pl.BlockSpec(memory_space=pltpu.MemorySpace.VMEM),
            pl.BlockSpec(memory_space=pltpu.MemorySpace.VMEM),
        ],
        out_specs=pl.BlockSpec(memory_space=pltpu.MemorySpace.VMEM),
        out_shape=jax.ShapeDtypeStruct((n, nclass), jnp.float32),
        scratch_shapes=[
            pltpu.VMEM((_NBUF, _BM, n), jnp.float32),
            pltpu.VMEM((n, nfeat), jnp.float32),
            pltpu.SemaphoreType.DMA((_NBUF, 2)),
        ],
        compiler_params=pltpu.CompilerParams(vmem_limit_bytes=100 * 2**20),
    )(adj_norm, x, W, b.reshape(1, nclass))


# speedup vs baseline: 1.0004x; 1.0004x over previous
import jax
import jax.numpy as jnp
from jax.experimental import pallas as pl
from jax.experimental.pallas import tpu as pltpu

_BM = 200
_NBUF = 5


def _sgc_kernel(a_hbm, x_ref, w_ref, b_ref, o_ref, bufs, y_ref, sems):
    n = x_ref.shape[0]
    nblk = n // _BM
    nsteps = 2 * nblk
    p1_reuse_end = nblk + _NBUF

    def block_of(s):
        return jnp.where(
            s < nblk, s,
            jnp.where(s < p1_reuse_end, s - _NBUF, s - p1_reuse_end))

    def dma(s):
        blk = block_of(s)
        slot = jax.lax.rem(s, _NBUF)
        return pltpu.make_async_copy(
            a_hbm.at[pl.ds(blk * _BM, _BM), :], bufs.at[slot], sems.at[slot])

    def needs_dma(s):
        return jnp.logical_or(s < nblk, s >= p1_reuse_end)

    for s in range(_NBUF):
        dma(s).start()

    def step(s, _):
        slot = jax.lax.rem(s, _NBUF)

        @pl.when(needs_dma(s))
        def _():
            dma(s).wait()

        a = bufs.at[slot][...]
        base = block_of(s) * _BM

        @pl.when(s < nblk)
        def _():
            y_ref[pl.ds(base, _BM), :] = jax.lax.dot_general(
                a, x_ref[...], (((1,), (0,)), ((), ())),
                preferred_element_type=jnp.float32)

        @pl.when(s >= nblk)
        def _():
            h = jax.lax.dot_general(
                a, y_ref[...], (((1,), (0,)), ((), ())),
                preferred_element_type=jnp.float32)
            logits = jax.lax.dot_general(
                h, w_ref[...], (((1,), (1,)), ((), ())),
                preferred_element_type=jnp.float32)
            logits = logits + b_ref[...]
            m = jnp.max(logits, axis=1, keepdims=True)
            shifted = logits - m
            lse = jnp.log(jnp.sum(jnp.exp(shifted), axis=1, keepdims=True))
            o_ref[pl.ds(base, _BM), :] = shifted - lse

        nxt = s + _NBUF

        @pl.when(jnp.logical_and(nxt < nsteps, needs_dma(nxt)))
        def _():
            dma(nxt).start()

        return _

    jax.lax.fori_loop(0, nsteps, step, None)


def kernel(x, adj_norm, W, b):
    n, nfeat = x.shape
    nclass = W.shape[0]

    return pl.pallas_call(
        _sgc_kernel,
        in_specs=[
            pl.BlockSpec(memory_space=pl.ANY),
            pl.BlockSpec(memory_space=pltpu.MemorySpace.VMEM),
            pl.BlockSpec(memory_space=pltpu.MemorySpace.VMEM),
            pl.BlockSpec(memory_space=pltpu.MemorySpace.VMEM),
        ],
        out_specs=pl.BlockSpec(memory_space=pltpu.MemorySpace.VMEM),
        out_shape=jax.ShapeDtypeStruct((n, nclass), jnp.float32),
        scratch_shapes=[
            pltpu.VMEM((_NBUF, _BM, n), jnp.float32),
            pltpu.VMEM((n, nfeat), jnp.float32),
            pltpu.SemaphoreType.DMA((_NBUF,)),
        ],
        compiler_params=pltpu.CompilerParams(vmem_limit_bytes=100 * 2**20),
    )(adj_norm, x, W, b.reshape(1, nclass))
